# baseline (speedup 1.0000x reference)
import functools
import math

import jax
import jax.numpy as jnp
from jax.experimental import pallas as pl

GS = 128
HEADS = 16
DH = 64
CROSS = math.log(0.125)
BETA_LF = 0.675
BETA_MF = 0.225
BETA_HF = 0.1
MQ = 512
QT = 256


def _qkv_pool_kernel(s, x_ref, srow_ref, pm_ref, st_ref, lt_ref, w_ref,
                     qkv_ref, pl_ref):
    xb = x_ref[0]
    nsub = MQ // GS
    if s == 1:
        px = xb
    else:
        R = GS * s
        w = jnp.clip(srow_ref[0], 1e-6, None)
        sg = st_ref[0]
        lg = lt_ref[0]
        if s == 2:
            plab = jnp.where(sg[0:1] >= sg[1:2], lg[0:1], lg[1:2])
        else:
            m01 = jnp.maximum(sg[0:1], sg[1:2])
            l01 = jnp.where(sg[0:1] >= sg[1:2], lg[0:1], lg[1:2])
            m23 = jnp.maximum(sg[2:3], sg[3:4])
            l23 = jnp.where(sg[2:3] >= sg[3:4], lg[2:3], lg[3:4])
            plab = jnp.where(m01 >= m23, l01, l23)
        pl_ref[0] = plab
        subs = []
        for j in range(nsub):
            wj = w[:, j * R:(j + 1) * R]
            wm = pm_ref[:] * jnp.broadcast_to(wj, (GS, R))
            den = jnp.sum(wm, axis=1, keepdims=True)
            wn = (wm / den).astype(jnp.bfloat16)
            subs.append(jnp.dot(wn, xb[j * R:(j + 1) * R, :],
                                preferred_element_type=jnp.float32
                                ).astype(jnp.bfloat16))
        px = jnp.concatenate(subs, axis=0)
    qkv_ref[0] = jnp.dot(px, w_ref[:],
                         preferred_element_type=jnp.float32
                         ).astype(jnp.bfloat16)


def _attn_kernel(fuse, *refs):
    if fuse:
        (q_ref, k0_ref, k1_ref, k2_ref, v0_ref, v1_ref, v2_ref,
         l0_ref, l1_ref, l2_ref,
         amf_ref, alf_ref, xres_ref, wp_ref, out_ref) = refs
    else:
        (q_ref, k0_ref, k1_ref, k2_ref, v0_ref, v1_ref, v2_ref,
         l0_ref, l1_ref, l2_ref, out_ref) = refs
    scale = 1.0 / math.sqrt(DH)
    q = (q_ref[0].astype(jnp.float32) * scale).astype(jnp.bfloat16)
    kA = jnp.concatenate([k0_ref[0], k1_ref[0]], axis=0)
    kB = jnp.concatenate([k1_ref[0], k2_ref[0]], axis=0)
    vA = jnp.concatenate([v0_ref[0], v1_ref[0]], axis=0)
    vB = jnp.concatenate([v1_ref[0], v2_ref[0]], axis=0)
    klA = jnp.concatenate([l0_ref[0], l1_ref[0]], axis=1)
    klB = jnp.concatenate([l1_ref[0], l2_ref[0]], axis=1)
    biasA = jnp.where(l0_ref[0].T == klA, 0.0, CROSS)
    biasB = jnp.where(l1_ref[0].T == klB, 0.0, CROSS)
    qA = q[:GS]
    qB = q[GS:]
    outsA = []
    outsB = []
    for h in range(HEADS):
        sl = slice(h * DH, (h + 1) * DH)
        for qh, kh, vh, bias, outs in (
                (qA, kA, vA, biasA, outsA), (qB, kB, vB, biasB, outsB)):
            lgt = jax.lax.dot_general(
                qh[:, sl], kh[:, sl], (((1,), (1,)), ((), ())),
                preferred_element_type=jnp.float32)
            p = jnp.exp(lgt + bias)
            s = jnp.sum(p, axis=-1, keepdims=True)
            o = jnp.dot(p.astype(jnp.bfloat16), vh[:, sl],
                        preferred_element_type=jnp.float32)
            outs.append(o / s)
    a = jnp.concatenate([jnp.concatenate(outsA, axis=1),
                         jnp.concatenate(outsB, axis=1)], axis=0)
    if fuse:
        C = a.shape[-1]
        amf = amf_ref[0].astype(jnp.float32)
        up2 = jnp.broadcast_to(amf[:, None, :], (GS, 2, C)).reshape(QT, C)
        alf = alf_ref[0].astype(jnp.float32)
        up4 = jnp.broadcast_to(alf[:, None, :], (64, 4, C)).reshape(QT, C)
        fused = BETA_HF * a + BETA_MF * up2 + BETA_LF * up4
        out_ref[0] = jnp.dot(fused.astype(jnp.bfloat16), wp_ref[:],
                             preferred_element_type=jnp.float32) + xres_ref[0]
    else:
        out_ref[0] = a.astype(jnp.bfloat16)


def _run_qkv(s, x_bf, scores, labels, wqkv, interpret=False):
    B, N, C = x_bf.shape
    np_ = N // s
    ngq = np_ // MQ
    R = MQ * s
    srow = scores.reshape(B * ngq, 1, R)
    st = scores.reshape(B * ngq, MQ, s).transpose(0, 2, 1)
    lt = labels.reshape(B * ngq, MQ, s).transpose(0, 2, 1).astype(jnp.int32)
    pmask = (jnp.arange(GS * s)[None, :] // s == jnp.arange(GS)[:, None]
             ).astype(jnp.float32)
    kern = functools.partial(_qkv_pool_kernel, s)
    qkv, plab = pl.pallas_call(
        kern,
        grid=(B, ngq),
        in_specs=[
            pl.BlockSpec((1, R, C), lambda b, i: (b, i, 0)),
            pl.BlockSpec((1, 1, R), lambda b, i, ngq=ngq: (b * ngq + i, 0, 0)),
            pl.BlockSpec((GS, GS * s), lambda b, i: (0, 0)),
            pl.BlockSpec((1, s, MQ), lambda b, i, ngq=ngq: (b * ngq + i, 0, 0)),
            pl.BlockSpec((1, s, MQ), lambda b, i, ngq=ngq: (b * ngq + i, 0, 0)),
            pl.BlockSpec((C, 3 * C), lambda b, i: (0, 0)),
        ],
        out_specs=[
            pl.BlockSpec((1, MQ, 3 * C), lambda b, i: (b, i, 0)),
            pl.BlockSpec((1, 1, MQ), lambda b, i, ngq=ngq: (b * ngq + i, 0, 0)),
        ],
        out_shape=[
            jax.ShapeDtypeStruct((B, np_, 3 * C), jnp.bfloat16),
            jax.ShapeDtypeStruct((B * ngq, 1, MQ), jnp.int32),
        ],
        interpret=interpret,
    )(x_bf, srow, pmask, st, lt, wqkv)
    return qkv, plab


def _run_attn(s, qkv, plab, fuse_args, interpret=False):
    B, np_, C3 = qkv.shape
    C = C3 // 3
    ng = np_ // GS
    ng2 = np_ // QT
    labs = plab.reshape(B * ng, 1, GS)
    nxt = lambda i: jnp.minimum(2 * i + 2, ng - 1)
    in_specs = [
        pl.BlockSpec((1, QT, C), lambda b, i: (b, i, 0)),
        pl.BlockSpec((1, GS, C), lambda b, i: (b, 2 * i, 1)),
        pl.BlockSpec((1, GS, C), lambda b, i: (b, 2 * i + 1, 1)),
        pl.BlockSpec((1, GS, C), lambda b, i: (b, nxt(i), 1)),
        pl.BlockSpec((1, GS, C), lambda b, i: (b, 2 * i, 2)),
        pl.BlockSpec((1, GS, C), lambda b, i: (b, 2 * i + 1, 2)),
        pl.BlockSpec((1, GS, C), lambda b, i: (b, nxt(i), 2)),
        pl.BlockSpec((1, 1, GS), lambda b, i, ng=ng: (b * ng + 2 * i, 0, 0)),
        pl.BlockSpec((1, 1, GS), lambda b, i, ng=ng: (b * ng + 2 * i + 1, 0, 0)),
        pl.BlockSpec((1, 1, GS), lambda b, i, ng=ng: (b * ng + nxt(i), 0, 0)),
    ]
    args = [qkv, qkv, qkv, qkv, qkv, qkv, qkv, labs, labs, labs]
    if fuse_args is not None:
        amf, alf, x, wp = fuse_args
        in_specs += [
            pl.BlockSpec((1, QT // 2, C), lambda b, i: (b, i, 0)),
            pl.BlockSpec((1, QT // 4, C), lambda b, i: (b, i, 0)),
            pl.BlockSpec((1, QT, C), lambda b, i: (b, i, 0)),
            pl.BlockSpec((C, C), lambda b, i: (0, 0)),
        ]
        args += [amf, alf, x, wp]
    kern = functools.partial(_attn_kernel, fuse_args is not None)
    out = pl.pallas_call(
        kern,
        grid=(B, ng2),
        in_specs=in_specs,
        out_specs=pl.BlockSpec((1, QT, C), lambda b, i: (b, i, 0)),
        out_shape=jax.ShapeDtypeStruct(
            (B, np_, C), jnp.float32 if fuse_args is not None else jnp.bfloat16),
        interpret=interpret,
    )(*args)
    return out


def _impl(x, labels, scores, Wq_hf, Wk_hf, Wv_hf, Wq_mf, Wk_mf, Wv_mf,
          Wq_lf, Wk_lf, Wv_lf, Wp, interpret=False):
    B, N, C = x.shape
    labels = labels.astype(jnp.int32)
    x_bf = x.astype(jnp.bfloat16)
    w_lf = jnp.concatenate([Wq_lf, Wk_lf, Wv_lf], axis=1).astype(jnp.bfloat16)
    w_mf = jnp.concatenate([Wq_mf, Wk_mf, Wv_mf], axis=1).astype(jnp.bfloat16)
    w_hf = jnp.concatenate([Wq_hf, Wk_hf, Wv_hf], axis=1).astype(jnp.bfloat16)
    Wp = Wp.astype(jnp.bfloat16)

    qkv_lf, pl_lf = _run_qkv(4, x_bf, scores, labels, w_lf, interpret)
    a_lf = _run_attn(4, qkv_lf, pl_lf, None, interpret)

    qkv_mf, pl_mf = _run_qkv(2, x_bf, scores, labels, w_mf, interpret)
    a_mf = _run_attn(2, qkv_mf, pl_mf, None, interpret)

    qkv_hf, _ = _run_qkv(1, x_bf, scores, labels, w_hf, interpret)
    pl_hf = labels.reshape(B * (N // GS), 1, GS)
    out = _run_attn(1, qkv_hf, pl_hf, (a_mf, a_lf, x, Wp), interpret)
    return out


def kernel(x, labels, scores, Wq_hf, Wk_hf, Wv_hf, Wq_mf, Wk_mf, Wv_mf,
           Wq_lf, Wk_lf, Wv_lf, Wp):
    return _impl(x, labels, scores, Wq_hf, Wk_hf, Wv_hf, Wq_mf, Wk_mf,
                 Wv_mf, Wq_lf, Wk_lf, Wv_lf, Wp)


# speedup vs baseline: 1.1179x; 1.1179x over previous
import functools
import math

import jax
import jax.numpy as jnp
from jax.experimental import pallas as pl

GS = 128
HEADS = 16
DH = 64
CROSS = math.log(0.125)
BETA_LF = 0.675
BETA_MF = 0.225
BETA_HF = 0.1
MQ = 512
QT = 256


def _qkv_pool_kernel(s, x_ref, srow_ref, pm_ref, st_ref, lt_ref, w_ref,
                     qkv_ref, pl_ref):
    xb = x_ref[0]
    nsub = MQ // GS
    if s == 1:
        px = xb
    else:
        R = GS * s
        w = jnp.clip(srow_ref[0], 1e-6, None)
        sg = st_ref[0]
        lg = lt_ref[0]
        if s == 2:
            plab = jnp.where(sg[0:1] >= sg[1:2], lg[0:1], lg[1:2])
        else:
            m01 = jnp.maximum(sg[0:1], sg[1:2])
            l01 = jnp.where(sg[0:1] >= sg[1:2], lg[0:1], lg[1:2])
            m23 = jnp.maximum(sg[2:3], sg[3:4])
            l23 = jnp.where(sg[2:3] >= sg[3:4], lg[2:3], lg[3:4])
            plab = jnp.where(m01 >= m23, l01, l23)
        pl_ref[0] = plab
        subs = []
        for j in range(nsub):
            wj = w[:, j * R:(j + 1) * R]
            wm = pm_ref[:] * jnp.broadcast_to(wj, (GS, R))
            den = jnp.sum(wm, axis=1, keepdims=True)
            wn = (wm / den).astype(jnp.bfloat16)
            subs.append(jnp.dot(wn, xb[j * R:(j + 1) * R, :],
                                preferred_element_type=jnp.float32
                                ).astype(jnp.bfloat16))
        px = jnp.concatenate(subs, axis=0)
    qkv_ref[0] = jnp.dot(px, w_ref[:],
                         preferred_element_type=jnp.float32
                         ).astype(jnp.bfloat16)


def _attn_kernel(fuse, *refs):
    if fuse:
        (q_ref, ks_ref, kn_ref, vs_ref, vn_ref, ls_ref, ln_ref,
         amf_ref, alf_ref, xres_ref, wp_ref, out_ref) = refs
    else:
        (q_ref, ks_ref, kn_ref, vs_ref, vn_ref, ls_ref, ln_ref,
         out_ref) = refs
    scale = 1.0 / math.sqrt(DH)
    q = (q_ref[0].astype(jnp.float32) * scale).astype(jnp.bfloat16)
    k = jnp.concatenate([ks_ref[0], kn_ref[0]], axis=0)
    v = jnp.concatenate([vs_ref[0], vn_ref[0]], axis=0)
    kl = jnp.concatenate([ls_ref[0], ln_ref[0]], axis=1)
    qlT = ls_ref[0].T
    bias = jnp.where(qlT == kl, 0.0, CROSS)
    outs = []
    for h in range(HEADS):
        sl = slice(h * DH, (h + 1) * DH)
        lg = jax.lax.dot_general(q[:, sl], k[:, sl], (((1,), (1,)), ((), ())),
                                 preferred_element_type=jnp.float32)
        p = jnp.exp(lg + bias)
        s = jnp.sum(p, axis=-1, keepdims=True)
        o = jnp.dot(p.astype(jnp.bfloat16), v[:, sl],
                    preferred_element_type=jnp.float32)
        outs.append(o / s)
    a = jnp.concatenate(outs, axis=1)
    if fuse:
        C = a.shape[-1]
        amf = amf_ref[0].astype(jnp.float32)
        up2 = jnp.broadcast_to(amf[:, None, :], (64, 2, C)).reshape(GS, C)
        alf = alf_ref[0].astype(jnp.float32)
        up4 = jnp.broadcast_to(alf[:, None, :], (32, 4, C)).reshape(GS, C)
        fused = BETA_HF * a + BETA_MF * up2 + BETA_LF * up4
        out_ref[0] = jnp.dot(fused.astype(jnp.bfloat16), wp_ref[:],
                             preferred_element_type=jnp.float32) + xres_ref[0]
    else:
        out_ref[0] = a.astype(jnp.bfloat16)


def _run_qkv(s, x_bf, scores, labels, wqkv, interpret=False):
    B, N, C = x_bf.shape
    np_ = N // s
    ngq = np_ // MQ
    R = MQ * s
    srow = scores.reshape(B * ngq, 1, R)
    st = scores.reshape(B * ngq, MQ, s).transpose(0, 2, 1)
    lt = labels.reshape(B * ngq, MQ, s).transpose(0, 2, 1).astype(jnp.int32)
    pmask = (jnp.arange(GS * s)[None, :] // s == jnp.arange(GS)[:, None]
             ).astype(jnp.float32)
    kern = functools.partial(_qkv_pool_kernel, s)
    qkv, plab = pl.pallas_call(
        kern,
        grid=(B, ngq),
        in_specs=[
            pl.BlockSpec((1, R, C), lambda b, i: (b, i, 0)),
            pl.BlockSpec((1, 1, R), lambda b, i, ngq=ngq: (b * ngq + i, 0, 0)),
            pl.BlockSpec((GS, GS * s), lambda b, i: (0, 0)),
            pl.BlockSpec((1, s, MQ), lambda b, i, ngq=ngq: (b * ngq + i, 0, 0)),
            pl.BlockSpec((1, s, MQ), lambda b, i, ngq=ngq: (b * ngq + i, 0, 0)),
            pl.BlockSpec((C, 3 * C), lambda b, i: (0, 0)),
        ],
        out_specs=[
            pl.BlockSpec((1, MQ, 3 * C), lambda b, i: (b, i, 0)),
            pl.BlockSpec((1, 1, MQ), lambda b, i, ngq=ngq: (b * ngq + i, 0, 0)),
        ],
        out_shape=[
            jax.ShapeDtypeStruct((B, np_, 3 * C), jnp.bfloat16),
            jax.ShapeDtypeStruct((B * ngq, 1, MQ), jnp.int32),
        ],
        interpret=interpret,
    )(x_bf, srow, pmask, st, lt, wqkv)
    return qkv, plab


def _run_attn(s, qkv, plab, fuse_args, interpret=False):
    B, np_, C3 = qkv.shape
    C = C3 // 3
    ng = np_ // GS
    labs = plab.reshape(B * ng, 1, GS)
    nxt = lambda i: jnp.minimum(i + 1, ng - 1)
    in_specs = [
        pl.BlockSpec((1, GS, C), lambda b, i: (b, i, 0)),
        pl.BlockSpec((1, GS, C), lambda b, i: (b, i, 1)),
        pl.BlockSpec((1, GS, C), lambda b, i: (b, nxt(i), 1)),
        pl.BlockSpec((1, GS, C), lambda b, i: (b, i, 2)),
        pl.BlockSpec((1, GS, C), lambda b, i: (b, nxt(i), 2)),
        pl.BlockSpec((1, 1, GS), lambda b, i, ng=ng: (b * ng + i, 0, 0)),
        pl.BlockSpec((1, 1, GS), lambda b, i, ng=ng: (b * ng + nxt(i), 0, 0)),
    ]
    args = [qkv, qkv, qkv, qkv, qkv, labs, labs]
    if fuse_args is not None:
        amf, alf, x, wp = fuse_args
        in_specs += [
            pl.BlockSpec((1, GS // 2, C), lambda b, i: (b, i, 0)),
            pl.BlockSpec((1, GS // 4, C), lambda b, i: (b, i, 0)),
            pl.BlockSpec((1, GS, C), lambda b, i: (b, i, 0)),
            pl.BlockSpec((C, C), lambda b, i: (0, 0)),
        ]
        args += [amf, alf, x, wp]
    kern = functools.partial(_attn_kernel, fuse_args is not None)
    out = pl.pallas_call(
        kern,
        grid=(B, ng),
        in_specs=in_specs,
        out_specs=pl.BlockSpec((1, GS, C), lambda b, i: (b, i, 0)),
        out_shape=jax.ShapeDtypeStruct(
            (B, np_, C), jnp.float32 if fuse_args is not None else jnp.bfloat16),
        interpret=interpret,
    )(*args)
    return out


def _impl(x, labels, scores, Wq_hf, Wk_hf, Wv_hf, Wq_mf, Wk_mf, Wv_mf,
          Wq_lf, Wk_lf, Wv_lf, Wp, interpret=False):
    B, N, C = x.shape
    labels = labels.astype(jnp.int32)
    x_bf = x.astype(jnp.bfloat16)
    w_lf = jnp.concatenate([Wq_lf, Wk_lf, Wv_lf], axis=1).astype(jnp.bfloat16)
    w_mf = jnp.concatenate([Wq_mf, Wk_mf, Wv_mf], axis=1).astype(jnp.bfloat16)
    w_hf = jnp.concatenate([Wq_hf, Wk_hf, Wv_hf], axis=1).astype(jnp.bfloat16)
    Wp = Wp.astype(jnp.bfloat16)

    qkv_lf, pl_lf = _run_qkv(4, x_bf, scores, labels, w_lf, interpret)
    a_lf = _run_attn(4, qkv_lf, pl_lf, None, interpret)

    qkv_mf, pl_mf = _run_qkv(2, x_bf, scores, labels, w_mf, interpret)
    a_mf = _run_attn(2, qkv_mf, pl_mf, None, interpret)

    qkv_hf, _ = _run_qkv(1, x_bf, scores, labels, w_hf, interpret)
    pl_hf = labels.reshape(B * (N // GS), 1, GS)
    out = _run_attn(1, qkv_hf, pl_hf, (a_mf, a_lf, x, Wp), interpret)
    return out


def kernel(x, labels, scores, Wq_hf, Wk_hf, Wv_hf, Wq_mf, Wk_mf, Wv_mf,
           Wq_lf, Wk_lf, Wv_lf, Wp):
    return _impl(x, labels, scores, Wq_hf, Wk_hf, Wv_hf, Wq_mf, Wk_mf,
                 Wv_mf, Wq_lf, Wk_lf, Wv_lf, Wp)


# speedup vs baseline: 1.2945x; 1.1579x over previous
import functools
import math

import jax
import jax.numpy as jnp
from jax.experimental import pallas as pl
from jax.experimental.pallas import tpu as pltpu

GS = 128
HEADS = 16
DH = 64
CROSS = math.log(0.125)
BETA_LF = 0.675
BETA_MF = 0.225
BETA_HF = 0.1
MQ = 512
NSUB = MQ // GS


def _attend(qt, ks, kn, vs, vn, ql, ln):
    k = jnp.concatenate([ks, kn], axis=0)
    v = jnp.concatenate([vs, vn], axis=0)
    kl = jnp.concatenate([ql, ln], axis=1)
    bias = jnp.where(ql.T == kl, 0.0, CROSS)
    outs = []
    for h in range(HEADS):
        sl = slice(h * DH, (h + 1) * DH)
        lg = jax.lax.dot_general(qt[:, sl], k[:, sl], (((1,), (1,)), ((), ())),
                                 preferred_element_type=jnp.float32)
        p = jnp.exp(lg + bias)
        s = jnp.sum(p, axis=-1, keepdims=True)
        o = jnp.dot(p.astype(jnp.bfloat16), v[:, sl],
                    preferred_element_type=jnp.float32)
        outs.append(o / s)
    return jnp.concatenate(outs, axis=1)


def _level_kernel(s, fuse, *refs):
    if s == 1:
        (x_ref, lt_ref, w_ref), rest = refs[:3], refs[3:]
    else:
        (x_ref, srow_ref, pm_ref, st_ref, lt_ref, w_ref), rest = \
            refs[:6], refs[6:]
    if fuse:
        (amf_ref, alf_ref, xres_ref, wp_ref, out_ref,
         kc_ref, vc_ref, lc_ref) = rest
    else:
        out_ref, kc_ref, vc_ref, lc_ref = rest
    j = pl.program_id(1)
    C = x_ref.shape[-1]

    xb = x_ref[0]
    if s == 1:
        px = xb
        plab = lt_ref[0]
    else:
        R = GS * s
        w = jnp.clip(srow_ref[0], 1e-6, None)
        sg = st_ref[0]
        lg_ = lt_ref[0]
        if s == 2:
            plab = jnp.where(sg[0:1] >= sg[1:2], lg_[0:1], lg_[1:2])
        else:
            m01 = jnp.maximum(sg[0:1], sg[1:2])
            l01 = jnp.where(sg[0:1] >= sg[1:2], lg_[0:1], lg_[1:2])
            m23 = jnp.maximum(sg[2:3], sg[3:4])
            l23 = jnp.where(sg[2:3] >= sg[3:4], lg_[2:3], lg_[3:4])
            plab = jnp.where(m01 >= m23, l01, l23)
        subs = []
        for t in range(NSUB):
            wt = w[:, t * R:(t + 1) * R]
            wm = pm_ref[:] * jnp.broadcast_to(wt, (GS, R))
            den = jnp.sum(wm, axis=1, keepdims=True)
            wn = (wm / den).astype(jnp.bfloat16)
            subs.append(jnp.dot(wn, xb[t * R:(t + 1) * R, :],
                                preferred_element_type=jnp.float32
                                ).astype(jnp.bfloat16))
        px = jnp.concatenate(subs, axis=0)

    qkv = jnp.dot(px, w_ref[:],
                  preferred_element_type=jnp.float32).astype(jnp.bfloat16)
    q = qkv[:, :C]
    k = qkv[:, C:2 * C]
    v = qkv[:, 2 * C:]

    k_carry = jnp.where(j == 0, k[(NSUB - 1) * GS:], kc_ref[:])
    v_carry = jnp.where(j == 0, v[(NSUB - 1) * GS:], vc_ref[:])
    l_carry = jnp.where(j == 0, plab[:, (NSUB - 1) * GS:], lc_ref[:])

    a_subs = []
    for t in range(NSUB):
        row = slice(t * GS, (t + 1) * GS)
        ql = plab[:, row]
        if t < NSUB - 1:
            nrow = slice((t + 1) * GS, (t + 2) * GS)
            a_subs.append(_attend(q[row], k[row], k[nrow], v[row], v[nrow],
                                  ql, plab[:, nrow]))
        else:
            a_subs.append(_attend(q[row], k[row], k_carry, v[row], v_carry,
                                  ql, l_carry))
    a = jnp.concatenate(a_subs, axis=0)

    kc_ref[:] = k[:GS]
    vc_ref[:] = v[:GS]
    lc_ref[:] = plab[:, :GS]

    if fuse:
        amf = amf_ref[0].astype(jnp.float32)
        up2 = jnp.broadcast_to(amf[:, None, :], (MQ // 2, 2, C)).reshape(MQ, C)
        alf = alf_ref[0].astype(jnp.float32)
        up4 = jnp.broadcast_to(alf[:, None, :], (MQ // 4, 4, C)).reshape(MQ, C)
        fused = BETA_HF * a + BETA_MF * up2 + BETA_LF * up4
        out_ref[0] = jnp.dot(fused.astype(jnp.bfloat16), wp_ref[:],
                             preferred_element_type=jnp.float32) + xres_ref[0]
    else:
        out_ref[0] = a.astype(jnp.bfloat16)


def _run_level(s, x_bf, scores, labels, wqkv, fuse_args, interpret=False):
    B, N, C = x_bf.shape
    np_ = N // s
    ngq = np_ // MQ
    R = MQ * s
    rev = lambda j: ngq - 1 - j
    in_specs = [pl.BlockSpec((1, R, C), lambda b, j: (b, rev(j), 0))]
    args = [x_bf]
    if s > 1:
        srow = scores.reshape(B * ngq, 1, R)
        st = scores.reshape(B * ngq, MQ, s).transpose(0, 2, 1)
        pmask = (jnp.arange(GS * s)[None, :] // s == jnp.arange(GS)[:, None]
                 ).astype(jnp.float32)
        in_specs += [
            pl.BlockSpec((1, 1, R), lambda b, j, g=ngq: (b * g + rev(j), 0, 0)),
            pl.BlockSpec((GS, GS * s), lambda b, j: (0, 0)),
            pl.BlockSpec((1, s, MQ), lambda b, j, g=ngq: (b * g + rev(j), 0, 0)),
        ]
        args += [srow, pmask, st]
    lt = labels.reshape(B * ngq, MQ, s).transpose(0, 2, 1)
    in_specs += [
        pl.BlockSpec((1, s, MQ), lambda b, j, g=ngq: (b * g + rev(j), 0, 0)),
        pl.BlockSpec((C, 3 * C), lambda b, j: (0, 0)),
    ]
    args += [lt, wqkv]
    if fuse_args is not None:
        amf, alf, x, wp = fuse_args
        in_specs += [
            pl.BlockSpec((1, MQ // 2, C), lambda b, j: (b, rev(j), 0)),
            pl.BlockSpec((1, MQ // 4, C), lambda b, j: (b, rev(j), 0)),
            pl.BlockSpec((1, MQ, C), lambda b, j: (b, rev(j), 0)),
            pl.BlockSpec((C, C), lambda b, j: (0, 0)),
        ]
        args += [amf, alf, x, wp]
    kern = functools.partial(_level_kernel, s, fuse_args is not None)
    out = pl.pallas_call(
        kern,
        grid=(B, ngq),
        in_specs=in_specs,
        out_specs=pl.BlockSpec((1, MQ, C), lambda b, j: (b, rev(j), 0)),
        out_shape=jax.ShapeDtypeStruct(
            (B, np_, C), jnp.float32 if fuse_args is not None else jnp.bfloat16),
        scratch_shapes=[
            pltpu.VMEM((GS, C), jnp.bfloat16),
            pltpu.VMEM((GS, C), jnp.bfloat16),
            pltpu.VMEM((1, GS), jnp.int32),
        ],
        interpret=interpret,
    )(*args)
    return out


def _impl(x, labels, scores, Wq_hf, Wk_hf, Wv_hf, Wq_mf, Wk_mf, Wv_mf,
          Wq_lf, Wk_lf, Wv_lf, Wp, interpret=False):
    B, N, C = x.shape
    labels = labels.astype(jnp.int32)
    x_bf = x.astype(jnp.bfloat16)
    scale = 1.0 / math.sqrt(DH)

    def wcat(wq, wk, wv):
        return jnp.concatenate([wq * scale, wk, wv], axis=1
                               ).astype(jnp.bfloat16)

    a_lf = _run_level(4, x_bf, scores, labels, wcat(Wq_lf, Wk_lf, Wv_lf),
                      None, interpret)
    a_mf = _run_level(2, x_bf, scores, labels, wcat(Wq_mf, Wk_mf, Wv_mf),
                      None, interpret)
    out = _run_level(1, x_bf, scores, labels, wcat(Wq_hf, Wk_hf, Wv_hf),
                     (a_mf, a_lf, x, Wp.astype(jnp.bfloat16)), interpret)
    return out


def kernel(x, labels, scores, Wq_hf, Wk_hf, Wv_hf, Wq_mf, Wk_mf, Wv_mf,
           Wq_lf, Wk_lf, Wv_lf, Wp):
    return _impl(x, labels, scores, Wq_hf, Wk_hf, Wv_hf, Wq_mf, Wk_mf,
                 Wv_mf, Wq_lf, Wk_lf, Wv_lf, Wp)
